# baseline (speedup 1.0000x reference)
import functools

import numpy as np

import jax
import jax.numpy as jnp
from jax.experimental import pallas as pl
from jax.experimental.pallas import tpu as pltpu

_IGNORE = -1
_VMEM_LIMIT = 48 * 1024 * 1024


def _upsample_matrix(src, dst, dst_pad):
    m = np.zeros((dst_pad, src), np.float32)
    d = np.arange(dst)
    s = np.maximum((d + 0.5) * (src / dst) - 0.5, 0.0)
    i0 = np.minimum(np.floor(s), src - 1).astype(np.int64)
    i1 = np.minimum(i0 + 1, src - 1)
    w1 = (s - i0).astype(np.float32)
    m[d, i0] += 1.0 - w1
    m[d, i1] += w1
    return m


def _ce_body(score_ref, wy_ref, wx_ref, lbl_ref, sum_ref, cnt_ref,
             ycs_ref, xs_ref, m_ref, *, num_classes, chunk, sub, imgs):
    wx = wx_ref[...]
    rb = wy_ref.shape[0]

    lsum = None
    lcnt = None
    for b in range(imgs):
        for cc in range(num_classes):
            ch = score_ref[b, cc].astype(jnp.bfloat16)
            yc = jnp.dot(wy_ref[...], ch, preferred_element_type=jnp.float32)
            ycs_ref[cc] = yc.astype(jnp.bfloat16)

        for r0 in range(0, rb, chunk):
            m = None
            for cc in range(num_classes):
                xc = jnp.dot(ycs_ref[cc, r0:r0 + chunk, :], wx,
                             preferred_element_type=jnp.float32)
                xs_ref[cc] = xc
                m = xc if m is None else jnp.maximum(m, xc)
            m_ref[...] = m

            for s0 in range(0, chunk, sub):
                t = lbl_ref[b, r0 + s0:r0 + s0 + sub, :]
                ms = m_ref[s0:s0 + sub, :]
                s = None
                picked = None
                for cc in range(num_classes):
                    x = xs_ref[cc, s0:s0 + sub, :]
                    e = jnp.exp(x - ms)
                    s = e if s is None else s + e
                    hit = jnp.where(t == cc, x, 0.0)
                    picked = hit if picked is None else picked + hit
                loss = ms + jnp.log(s) - picked
                valid = t != _IGNORE
                ls = jnp.sum(jnp.where(valid, loss, 0.0), axis=0, keepdims=True)
                lc = jnp.sum(valid.astype(jnp.float32), axis=0, keepdims=True)
                lsum = ls if lsum is None else lsum + ls
                lcnt = lc if lcnt is None else lcnt + lc
    sum_ref[0] = lsum
    cnt_ref[0] = lcnt


def kernel(score, target):
    n, c, hs, ws = score.shape
    _, h, w = target.shape

    rb = min(h, 512)
    bands = pl.cdiv(h, rb)
    h_pad = bands * rb
    if h_pad != h:
        target = jnp.pad(target, ((0, 0), (0, h_pad - h), (0, 0)),
                         constant_values=_IGNORE)

    wy = jnp.asarray(_upsample_matrix(hs, h, h_pad)).astype(jnp.bfloat16)
    wx = jnp.asarray(_upsample_matrix(ws, w, w).T).astype(jnp.bfloat16)
    work = n * bands

    chunk = 64 if rb % 64 == 0 else rb
    sub = 16 if chunk % 16 == 0 else chunk
    imgs = 2 if bands == 1 and n % 2 == 0 else 1
    steps = work // imgs

    body = functools.partial(_ce_body, num_classes=c, chunk=chunk, sub=sub,
                             imgs=imgs)
    part_sum, part_cnt = pl.pallas_call(
        body,
        out_shape=(jax.ShapeDtypeStruct((steps, 1, w), jnp.float32),
                   jax.ShapeDtypeStruct((steps, 1, w), jnp.float32)),
        grid_spec=pltpu.PrefetchScalarGridSpec(
            num_scalar_prefetch=0,
            grid=(steps,),
            in_specs=[
                pl.BlockSpec((imgs, c, hs, ws),
                             lambda i: (i // bands, 0, 0, 0)),
                pl.BlockSpec((rb, hs), lambda i: (i % bands, 0)),
                pl.BlockSpec((ws, w), lambda i: (0, 0)),
                pl.BlockSpec((imgs, rb, w), lambda i: (i // bands, i % bands, 0)),
            ],
            out_specs=[
                pl.BlockSpec((1, 1, w), lambda i: (i, 0, 0)),
                pl.BlockSpec((1, 1, w), lambda i: (i, 0, 0)),
            ],
            scratch_shapes=[pltpu.VMEM((c, rb, hs), jnp.bfloat16),
                            pltpu.VMEM((c, chunk, w), jnp.float32),
                            pltpu.VMEM((chunk, w), jnp.float32)],
        ),
        compiler_params=pltpu.CompilerParams(
            dimension_semantics=("parallel",),
            vmem_limit_bytes=_VMEM_LIMIT),
    )(score, wy, wx, target)

    return (jnp.sum(part_sum) / jnp.sum(part_cnt)).astype(jnp.float32)


# speedup vs baseline: 1.0592x; 1.0592x over previous
import functools

import numpy as np

import jax
import jax.numpy as jnp
from jax.experimental import pallas as pl
from jax.experimental.pallas import tpu as pltpu

_IGNORE = -1
_VMEM_LIMIT = 48 * 1024 * 1024


def _upsample_matrix(src, dst, dst_pad):
    m = np.zeros((dst_pad, src), np.float32)
    d = np.arange(dst)
    s = np.maximum((d + 0.5) * (src / dst) - 0.5, 0.0)
    i0 = np.minimum(np.floor(s), src - 1).astype(np.int64)
    i1 = np.minimum(i0 + 1, src - 1)
    w1 = (s - i0).astype(np.float32)
    m[d, i0] += 1.0 - w1
    m[d, i1] += w1
    return m


def _ce_body(score_ref, wy_ref, wx_ref, lbl_ref, out_ref,
             ycs_ref, xs_ref, m_ref, asum_ref, acnt_ref, *,
             num_classes, chunk, sub, imgs, steps):
    wx = wx_ref[...]
    rb = wy_ref.shape[0]
    step = pl.program_id(0)

    @pl.when(step == 0)
    def _():
        asum_ref[...] = jnp.zeros_like(asum_ref)
        acnt_ref[...] = jnp.zeros_like(acnt_ref)

    lsum = None
    lcnt = None
    for b in range(imgs):
        for cc in range(num_classes):
            ch = score_ref[b, cc].astype(jnp.bfloat16)
            yc = jnp.dot(wy_ref[...], ch, preferred_element_type=jnp.float32)
            ycs_ref[cc] = yc.astype(jnp.bfloat16)

        for r0 in range(0, rb, chunk):
            m = None
            for cc in range(num_classes):
                xc = jnp.dot(ycs_ref[cc, r0:r0 + chunk, :], wx,
                             preferred_element_type=jnp.float32)
                xs_ref[cc] = xc
                m = xc if m is None else jnp.maximum(m, xc)
            m_ref[...] = m

            for s0 in range(0, chunk, sub):
                t = lbl_ref[b, r0 + s0:r0 + s0 + sub, :]
                ms = m_ref[s0:s0 + sub, :]
                s = None
                picked = None
                for cc in range(num_classes):
                    x = xs_ref[cc, s0:s0 + sub, :]
                    e = jnp.exp(x - ms)
                    s = e if s is None else s + e
                    hit = jnp.where(t == cc, x, 0.0)
                    picked = hit if picked is None else picked + hit
                loss = ms + jnp.log(s) - picked
                valid = t != _IGNORE
                ls = jnp.sum(jnp.where(valid, loss, 0.0), axis=0, keepdims=True)
                lc = jnp.sum(valid.astype(jnp.float32), axis=0, keepdims=True)
                lsum = ls if lsum is None else lsum + ls
                lcnt = lc if lcnt is None else lcnt + lc
    asum_ref[...] = asum_ref[...] + lsum
    acnt_ref[...] = acnt_ref[...] + lcnt

    @pl.when(step == steps - 1)
    def _():
        total = jnp.sum(asum_ref[...]) / jnp.sum(acnt_ref[...])
        out_ref[...] = jnp.full((1, 1), total, jnp.float32)


def kernel(score, target):
    n, c, hs, ws = score.shape
    _, h, w = target.shape

    rb = min(h, 512)
    bands = pl.cdiv(h, rb)
    h_pad = bands * rb
    if h_pad != h:
        target = jnp.pad(target, ((0, 0), (0, h_pad - h), (0, 0)),
                         constant_values=_IGNORE)

    wy = jnp.asarray(_upsample_matrix(hs, h, h_pad)).astype(jnp.bfloat16)
    wx = jnp.asarray(_upsample_matrix(ws, w, w).T).astype(jnp.bfloat16)
    work = n * bands

    chunk = 64 if rb % 64 == 0 else rb
    sub = 16 if chunk % 16 == 0 else chunk
    imgs = 1
    steps = work // imgs

    body = functools.partial(_ce_body, num_classes=c, chunk=chunk, sub=sub,
                             imgs=imgs, steps=steps)
    out = pl.pallas_call(
        body,
        out_shape=jax.ShapeDtypeStruct((1, 1), jnp.float32),
        grid_spec=pltpu.PrefetchScalarGridSpec(
            num_scalar_prefetch=0,
            grid=(steps,),
            in_specs=[
                pl.BlockSpec((imgs, c, hs, ws),
                             lambda i: (i // bands, 0, 0, 0)),
                pl.BlockSpec((rb, hs), lambda i: (i % bands, 0)),
                pl.BlockSpec((ws, w), lambda i: (0, 0)),
                pl.BlockSpec((imgs, rb, w), lambda i: (i // bands, i % bands, 0)),
            ],
            out_specs=pl.BlockSpec((1, 1), lambda i: (0, 0)),
            scratch_shapes=[pltpu.VMEM((c, rb, hs), jnp.bfloat16),
                            pltpu.VMEM((c, chunk, w), jnp.float32),
                            pltpu.VMEM((chunk, w), jnp.float32),
                            pltpu.VMEM((1, w), jnp.float32),
                            pltpu.VMEM((1, w), jnp.float32)],
        ),
        compiler_params=pltpu.CompilerParams(
            dimension_semantics=("arbitrary",),
            vmem_limit_bytes=_VMEM_LIMIT),
    )(score, wy, wx, target)

    return out[0, 0]


# speedup vs baseline: 1.0828x; 1.0223x over previous
import functools

import numpy as np

import jax
import jax.numpy as jnp
from jax.experimental import pallas as pl
from jax.experimental.pallas import tpu as pltpu

_IGNORE = -1
_VMEM_LIMIT = 48 * 1024 * 1024
_LN2 = float(np.log(2.0))
_LOG2E = float(np.log2(np.e))


def _upsample_matrix(src, dst, dst_pad):
    m = np.zeros((dst_pad, src), np.float32)
    d = np.arange(dst)
    s = np.maximum((d + 0.5) * (src / dst) - 0.5, 0.0)
    i0 = np.minimum(np.floor(s), src - 1).astype(np.int64)
    i1 = np.minimum(i0 + 1, src - 1)
    w1 = (s - i0).astype(np.float32)
    m[d, i0] += 1.0 - w1
    m[d, i1] += w1
    return m


def _ce_body(score_ref, wy_ref, wx_ref, lbl_ref, out_ref,
             ycs_ref, xs_ref, m_ref, asum_ref, acnt_ref, *,
             num_classes, chunk, sub, imgs, steps):
    wx = wx_ref[...]
    rb = wy_ref.shape[0]
    step = pl.program_id(0)

    @pl.when(step == 0)
    def _():
        asum_ref[...] = jnp.zeros_like(asum_ref)
        acnt_ref[...] = jnp.zeros_like(acnt_ref)

    lsum = None
    lcnt = None
    for b in range(imgs):
        for cc in range(num_classes):
            ch = score_ref[b, cc].astype(jnp.bfloat16)
            yc = jnp.dot(wy_ref[...], ch, preferred_element_type=jnp.float32)
            ycs_ref[cc] = yc.astype(jnp.bfloat16)

        for r0 in range(0, rb, chunk):
            m = None
            for cc in range(num_classes):
                xc = jnp.dot(ycs_ref[cc, r0:r0 + chunk, :], wx,
                             preferred_element_type=jnp.float32)
                xs_ref[cc] = xc
                m = xc if m is None else jnp.maximum(m, xc)
            m_ref[...] = m

            for s0 in range(0, chunk, sub):
                t = lbl_ref[b, r0 + s0:r0 + s0 + sub, :]
                ms = m_ref[s0:s0 + sub, :]
                s = None
                picked = None
                for cc in range(num_classes):
                    x = xs_ref[cc, s0:s0 + sub, :]
                    e = jnp.exp2(x - ms)
                    s = e if s is None else s + e
                    hit = jnp.where(t == cc, x, 0.0)
                    picked = hit if picked is None else picked + hit
                loss = _LN2 * (ms - picked) + jnp.log(s)
                valid = t != _IGNORE
                ls = jnp.sum(jnp.where(valid, loss, 0.0), axis=0, keepdims=True)
                lc = jnp.sum(valid.astype(jnp.float32), axis=0, keepdims=True)
                lsum = ls if lsum is None else lsum + ls
                lcnt = lc if lcnt is None else lcnt + lc
    asum_ref[...] = asum_ref[...] + lsum
    acnt_ref[...] = acnt_ref[...] + lcnt

    @pl.when(step == steps - 1)
    def _():
        total = jnp.sum(asum_ref[...]) / jnp.sum(acnt_ref[...])
        out_ref[...] = jnp.full((1, 1), total, jnp.float32)


def kernel(score, target):
    n, c, hs, ws = score.shape
    _, h, w = target.shape

    rb = min(h, 512)
    bands = pl.cdiv(h, rb)
    h_pad = bands * rb
    if h_pad != h:
        target = jnp.pad(target, ((0, 0), (0, h_pad - h), (0, 0)),
                         constant_values=_IGNORE)

    wy = jnp.asarray(_upsample_matrix(hs, h, h_pad)).astype(jnp.bfloat16)
    wx = jnp.asarray(_upsample_matrix(ws, w, w).T * _LOG2E).astype(jnp.bfloat16)
    work = n * bands

    chunk = 64 if rb % 64 == 0 else rb
    sub = 16 if chunk % 16 == 0 else chunk
    imgs = 1
    steps = work // imgs

    body = functools.partial(_ce_body, num_classes=c, chunk=chunk, sub=sub,
                             imgs=imgs, steps=steps)
    out = pl.pallas_call(
        body,
        out_shape=jax.ShapeDtypeStruct((1, 1), jnp.float32),
        grid_spec=pltpu.PrefetchScalarGridSpec(
            num_scalar_prefetch=0,
            grid=(steps,),
            in_specs=[
                pl.BlockSpec((imgs, c, hs, ws),
                             lambda i: (i // bands, 0, 0, 0)),
                pl.BlockSpec((rb, hs), lambda i: (i % bands, 0)),
                pl.BlockSpec((ws, w), lambda i: (0, 0)),
                pl.BlockSpec((imgs, rb, w), lambda i: (i // bands, i % bands, 0)),
            ],
            out_specs=pl.BlockSpec((1, 1), lambda i: (0, 0)),
            scratch_shapes=[pltpu.VMEM((c, rb, hs), jnp.bfloat16),
                            pltpu.VMEM((c, chunk, w), jnp.float32),
                            pltpu.VMEM((chunk, w), jnp.float32),
                            pltpu.VMEM((1, w), jnp.float32),
                            pltpu.VMEM((1, w), jnp.float32)],
        ),
        compiler_params=pltpu.CompilerParams(
            dimension_semantics=("arbitrary",),
            vmem_limit_bytes=_VMEM_LIMIT),
    )(score, wy, wx, target)

    return out[0, 0]


# speedup vs baseline: 1.1266x; 1.0405x over previous
import functools

import numpy as np

import jax
import jax.numpy as jnp
from jax.experimental import pallas as pl
from jax.experimental.pallas import tpu as pltpu

_IGNORE = -1
_VMEM_LIMIT = 48 * 1024 * 1024
_LN2 = float(np.log(2.0))
_LOG2E = float(np.log2(np.e))


def _upsample_matrix(src, dst, dst_pad):
    m = np.zeros((dst_pad, src), np.float32)
    d = np.arange(dst)
    s = np.maximum((d + 0.5) * (src / dst) - 0.5, 0.0)
    i0 = np.minimum(np.floor(s), src - 1).astype(np.int64)
    i1 = np.minimum(i0 + 1, src - 1)
    w1 = (s - i0).astype(np.float32)
    m[d, i0] += 1.0 - w1
    m[d, i1] += w1
    return m


def _ce_body(score_ref, wy_ref, wx_ref, lbl_ref, out_ref,
             ycs_ref, xs_ref, m_ref, asum_ref, acnt_ref, *,
             num_classes, chunk, sub, imgs, steps):
    wx = wx_ref[...]
    rb = wy_ref.shape[0]
    step = pl.program_id(0)

    @pl.when(step == 0)
    def _():
        asum_ref[...] = jnp.zeros_like(asum_ref)
        acnt_ref[...] = jnp.zeros_like(acnt_ref)

    lsum = None
    lcnt = None
    for b in range(imgs):
        for cc in range(num_classes):
            ch = score_ref[b, cc].astype(jnp.bfloat16)
            yc = jnp.dot(wy_ref[...], ch, preferred_element_type=jnp.float32)
            ycs_ref[cc] = yc.astype(jnp.bfloat16)

        for r0 in range(0, rb, chunk):
            m = None
            for cc in range(num_classes):
                xc = jnp.dot(ycs_ref[cc, r0:r0 + chunk, :], wx,
                             preferred_element_type=jnp.float32)
                xs_ref[cc] = xc
                m = xc if m is None else jnp.maximum(m, xc)
            m_ref[...] = m

            for s0 in range(0, chunk, sub):
                t = lbl_ref[b, r0 + s0:r0 + s0 + sub, :]
                ms = m_ref[s0:s0 + sub, :]
                s = None
                picked = None
                for cc in range(num_classes):
                    x = xs_ref[cc, s0:s0 + sub, :]
                    e = jnp.exp2(x - ms)
                    s = e if s is None else s + e
                    hit = jnp.where(t == cc, x, 0.0)
                    picked = hit if picked is None else picked + hit
                loss = _LN2 * (ms - picked) + jnp.log(s)
                valid = t != _IGNORE
                ls = jnp.sum(jnp.where(valid, loss, 0.0), axis=0, keepdims=True)
                lc = jnp.sum(valid.astype(jnp.float32), axis=0, keepdims=True)
                lsum = ls if lsum is None else lsum + ls
                lcnt = lc if lcnt is None else lcnt + lc
    asum_ref[...] = asum_ref[...] + lsum
    acnt_ref[...] = acnt_ref[...] + lcnt

    @pl.when(step == steps - 1)
    def _():
        total = jnp.sum(asum_ref[...]) / jnp.sum(acnt_ref[...])
        out_ref[...] = jnp.full((1, 1), total, jnp.float32)


def kernel(score, target):
    n, c, hs, ws = score.shape
    _, h, w = target.shape

    rb = min(h, 512)
    bands = pl.cdiv(h, rb)
    h_pad = bands * rb
    if h_pad != h:
        target = jnp.pad(target, ((0, 0), (0, h_pad - h), (0, 0)),
                         constant_values=_IGNORE)

    wy = jnp.asarray(_upsample_matrix(hs, h, h_pad)).astype(jnp.bfloat16)
    wx = jnp.asarray(_upsample_matrix(ws, w, w).T * _LOG2E).astype(jnp.bfloat16)
    work = n * bands

    chunk = 32 if rb % 32 == 0 else rb
    sub = 16 if chunk % 16 == 0 else chunk
    imgs = 1
    steps = work // imgs

    body = functools.partial(_ce_body, num_classes=c, chunk=chunk, sub=sub,
                             imgs=imgs, steps=steps)
    out = pl.pallas_call(
        body,
        out_shape=jax.ShapeDtypeStruct((1, 1), jnp.float32),
        grid_spec=pltpu.PrefetchScalarGridSpec(
            num_scalar_prefetch=0,
            grid=(steps,),
            in_specs=[
                pl.BlockSpec((imgs, c, hs, ws),
                             lambda i: (i // bands, 0, 0, 0)),
                pl.BlockSpec((rb, hs), lambda i: (i % bands, 0)),
                pl.BlockSpec((ws, w), lambda i: (0, 0)),
                pl.BlockSpec((imgs, rb, w), lambda i: (i // bands, i % bands, 0)),
            ],
            out_specs=pl.BlockSpec((1, 1), lambda i: (0, 0)),
            scratch_shapes=[pltpu.VMEM((c, rb, hs), jnp.bfloat16),
                            pltpu.VMEM((c, chunk, w), jnp.float32),
                            pltpu.VMEM((chunk, w), jnp.float32),
                            pltpu.VMEM((1, w), jnp.float32),
                            pltpu.VMEM((1, w), jnp.float32)],
        ),
        compiler_params=pltpu.CompilerParams(
            dimension_semantics=("arbitrary",),
            vmem_limit_bytes=_VMEM_LIMIT),
    )(score, wy, wx, target)

    return out[0, 0]


# speedup vs baseline: 1.2103x; 1.0743x over previous
import functools

import numpy as np

import jax
import jax.numpy as jnp
from jax.experimental import pallas as pl
from jax.experimental.pallas import tpu as pltpu

_IGNORE = -1
_VMEM_LIMIT = 48 * 1024 * 1024
_LN2 = float(np.log(2.0))
_LOG2E = float(np.log2(np.e))


def _upsample_matrix(src, dst, dst_pad):
    m = np.zeros((dst_pad, src), np.float32)
    d = np.arange(dst)
    s = np.maximum((d + 0.5) * (src / dst) - 0.5, 0.0)
    i0 = np.minimum(np.floor(s), src - 1).astype(np.int64)
    i1 = np.minimum(i0 + 1, src - 1)
    w1 = (s - i0).astype(np.float32)
    m[d, i0] += 1.0 - w1
    m[d, i1] += w1
    return m


def _ce_body(score_ref, wy_ref, wx_ref, lbl_ref, out_ref,
             ycs_ref, xs_ref, m_ref, asum_ref, acnt_ref, *,
             num_classes, chunk, sub, imgs, steps):
    wx = wx_ref[...]
    rb = wy_ref.shape[0]
    step = pl.program_id(0)

    @pl.when(step == 0)
    def _():
        asum_ref[...] = jnp.zeros_like(asum_ref)
        acnt_ref[...] = jnp.zeros_like(acnt_ref)

    lsum = None
    lcnt = None
    for b in range(imgs):
        for cc in range(num_classes):
            ch = score_ref[b, cc].astype(jnp.bfloat16)
            yc = jnp.dot(wy_ref[...], ch, preferred_element_type=jnp.float32)
            ycs_ref[cc] = yc.astype(jnp.bfloat16)

        for r0 in range(0, rb, chunk):
            m = None
            for cc in range(num_classes):
                xc = jnp.dot(ycs_ref[cc, r0:r0 + chunk, :], wx,
                             preferred_element_type=jnp.float32)
                xs_ref[cc] = xc
                m = xc if m is None else jnp.maximum(m, xc)
            if sub != chunk:
                m_ref[...] = m

            for s0 in range(0, chunk, sub):
                t = lbl_ref[b, r0 + s0:r0 + s0 + sub, :]
                ms = m if sub == chunk else m_ref[s0:s0 + sub, :]
                s = None
                picked = None
                for cc in range(num_classes):
                    x = xs_ref[cc, s0:s0 + sub, :]
                    e = jnp.exp2(x - ms)
                    s = e if s is None else s + e
                    hit = jnp.where(t == cc, x, 0.0)
                    picked = hit if picked is None else picked + hit
                loss = _LN2 * (ms - picked) + jnp.log(s)
                valid = t != _IGNORE
                ls = jnp.sum(jnp.where(valid, loss, 0.0), axis=0, keepdims=True)
                lc = jnp.sum(valid.astype(jnp.float32), axis=0, keepdims=True)
                lsum = ls if lsum is None else lsum + ls
                lcnt = lc if lcnt is None else lcnt + lc
    asum_ref[...] = asum_ref[...] + lsum
    acnt_ref[...] = acnt_ref[...] + lcnt

    @pl.when(step == steps - 1)
    def _():
        total = jnp.sum(asum_ref[...]) / jnp.sum(acnt_ref[...])
        out_ref[...] = jnp.full((1, 1), total, jnp.float32)


def kernel(score, target):
    n, c, hs, ws = score.shape
    _, h, w = target.shape

    rb = min(h, 512)
    bands = pl.cdiv(h, rb)
    h_pad = bands * rb
    if h_pad != h:
        target = jnp.pad(target, ((0, 0), (0, h_pad - h), (0, 0)),
                         constant_values=_IGNORE)

    wy = jnp.asarray(_upsample_matrix(hs, h, h_pad)).astype(jnp.bfloat16)
    wx = jnp.asarray(_upsample_matrix(ws, w, w).T * _LOG2E).astype(jnp.bfloat16)
    work = n * bands

    chunk = 32 if rb % 32 == 0 else rb
    sub = 8 if chunk % 8 == 0 else chunk
    imgs = 1
    steps = work // imgs

    body = functools.partial(_ce_body, num_classes=c, chunk=chunk, sub=sub,
                             imgs=imgs, steps=steps)
    out = pl.pallas_call(
        body,
        out_shape=jax.ShapeDtypeStruct((1, 1), jnp.float32),
        grid_spec=pltpu.PrefetchScalarGridSpec(
            num_scalar_prefetch=0,
            grid=(steps,),
            in_specs=[
                pl.BlockSpec((imgs, c, hs, ws),
                             lambda i: (i // bands, 0, 0, 0)),
                pl.BlockSpec((rb, hs), lambda i: (i % bands, 0)),
                pl.BlockSpec((ws, w), lambda i: (0, 0)),
                pl.BlockSpec((imgs, rb, w), lambda i: (i // bands, i % bands, 0)),
            ],
            out_specs=pl.BlockSpec((1, 1), lambda i: (0, 0)),
            scratch_shapes=[pltpu.VMEM((c, rb, hs), jnp.bfloat16),
                            pltpu.VMEM((c, chunk, w), jnp.float32),
                            pltpu.VMEM((chunk, w), jnp.float32),
                            pltpu.VMEM((1, w), jnp.float32),
                            pltpu.VMEM((1, w), jnp.float32)],
        ),
        compiler_params=pltpu.CompilerParams(
            dimension_semantics=("arbitrary",),
            vmem_limit_bytes=_VMEM_LIMIT),
    )(score, wy, wx, target)

    return out[0, 0]
